# Initial kernel scaffold; baseline (speedup 1.0000x reference)
import jax
import jax.numpy as jnp
from jax.experimental import pallas as pl


def kernel(x, edge_index, batch, W1a, b1a, W1b, b1b, W2a, b2a, W2b, b2b, W3a, b3a, W3b, b3b, Wf, bf):
    raise NotImplementedError("write your pallas kernel here")


# speedup vs baseline: 8.1612x; 8.1612x over previous
import functools

import jax
import jax.numpy as jnp
from jax import lax
from jax.experimental import pallas as pl
from jax.experimental.pallas import tpu as pltpu
from jax.experimental.pallas import tpu_sc as plsc

N, E, F, H, G = 10000, 320000, 128, 64, 128

NUM_CORES = 2
NUM_SUBCORES = 16
NUM_WORKERS = NUM_CORES * NUM_SUBCORES
EDGES_PER_TILE = E // NUM_WORKERS
CHUNK = 80
CHUNKS = EDGES_PER_TILE // CHUNK
ROWS_BIG = 632
ROWS_LAST = N - 15 * ROWS_BIG

_HIGH = jax.lax.Precision.HIGHEST


def _agg_body(y_hbm, srcs_hbm, dsts_hbm, zeros_hbm, out_hbm,
              src_v, dst_v, rows_v, acc_sh, sem):
    c = lax.axis_index("c")
    s = lax.axis_index("s")
    base = s * ROWS_BIG
    pltpu.sync_copy(srcs_hbm.at[c].at[s], src_v)
    pltpu.sync_copy(dsts_hbm.at[c].at[s], dst_v)

    @pl.when(s < 15)
    def _():
        pltpu.sync_copy(zeros_hbm, acc_sh.at[pl.ds(base, ROWS_BIG)])

    @pl.when(s == 15)
    def _():
        pltpu.sync_copy(zeros_hbm.at[pl.ds(0, ROWS_LAST)],
                        acc_sh.at[pl.ds(15 * ROWS_BIG, ROWS_LAST)])

    plsc.subcore_barrier()

    def step(j, carry):
        pltpu.async_copy(y_hbm.at[src_v.at[j]], rows_v, sem).wait()
        pltpu.sync_copy(rows_v, acc_sh.at[dst_v.at[j]], add=True)
        return carry

    lax.fori_loop(0, CHUNKS, step, 0)
    plsc.subcore_barrier()

    @pl.when(s < 15)
    def _():
        pltpu.sync_copy(acc_sh.at[pl.ds(base, ROWS_BIG)],
                        out_hbm.at[c].at[pl.ds(base, ROWS_BIG)])

    @pl.when(s == 15)
    def _():
        pltpu.sync_copy(acc_sh.at[pl.ds(15 * ROWS_BIG, ROWS_LAST)],
                        out_hbm.at[c].at[pl.ds(15 * ROWS_BIG, ROWS_LAST)])


_agg = pl.kernel(
    _agg_body,
    out_type=jax.ShapeDtypeStruct((NUM_CORES, N, H), jnp.float32),
    mesh=plsc.VectorSubcoreMesh(core_axis_name="c", subcore_axis_name="s"),
    scratch_types=[
        pltpu.VMEM((CHUNKS, CHUNK), jnp.int32),
        pltpu.VMEM((CHUNKS, CHUNK), jnp.int32),
        pltpu.VMEM((CHUNK, H), jnp.float32),
        pltpu.VMEM_SHARED((N, H), jnp.float32),
        pltpu.SemaphoreType.DMA,
    ],
    compiler_params=pltpu.CompilerParams(use_tc_tiling_on_sc=False),
)


_BLK = 2000


def _proj_body(x_ref, w_ref, o_ref):
    o_ref[...] = jnp.dot(x_ref[...], w_ref[...],
                         preferred_element_type=jnp.float32, precision=_HIGH)


_proj = pl.pallas_call(
    _proj_body,
    grid=(N // _BLK,),
    in_specs=[
        pl.BlockSpec((_BLK, F), lambda i: (i, 0)),
        pl.BlockSpec((F, H), lambda i: (0, 0)),
    ],
    out_specs=pl.BlockSpec((_BLK, H), lambda i: (i, 0)),
    out_shape=jax.ShapeDtypeStruct((N, H), jnp.float32),
)


def _layer1_body(y_ref, p0_ref, p1_ref, ba_ref, wb_ref, bb_ref, o_ref):
    t = y_ref[...] + p0_ref[...] + p1_ref[...] + ba_ref[...]
    t = jnp.maximum(t, 0.0)
    t = jnp.dot(t, wb_ref[...], preferred_element_type=jnp.float32,
                precision=_HIGH) + bb_ref[...]
    o_ref[...] = jnp.maximum(t, 0.0)


_layer1 = pl.pallas_call(
    _layer1_body,
    grid=(N // _BLK,),
    in_specs=[
        pl.BlockSpec((_BLK, H), lambda i: (i, 0)),
        pl.BlockSpec((_BLK, H), lambda i: (i, 0)),
        pl.BlockSpec((_BLK, H), lambda i: (i, 0)),
        pl.BlockSpec((1, H), lambda i: (0, 0)),
        pl.BlockSpec((H, H), lambda i: (0, 0)),
        pl.BlockSpec((1, H), lambda i: (0, 0)),
    ],
    out_specs=pl.BlockSpec((_BLK, H), lambda i: (i, 0)),
    out_shape=jax.ShapeDtypeStruct((N, H), jnp.float32),
)


def _layer23_body(h_ref, p0_ref, p1_ref, wa_ref, ba_ref, wb_ref, bb_ref,
                  o_ref, *, final_relu):
    t = h_ref[...] + p0_ref[...] + p1_ref[...]
    t = jnp.dot(t, wa_ref[...], preferred_element_type=jnp.float32,
                precision=_HIGH) + ba_ref[...]
    t = jnp.maximum(t, 0.0)
    t = jnp.dot(t, wb_ref[...], preferred_element_type=jnp.float32,
                precision=_HIGH) + bb_ref[...]
    o_ref[...] = jnp.maximum(t, 0.0) if final_relu else t


def _make_layer23(final_relu):
    return pl.pallas_call(
        functools.partial(_layer23_body, final_relu=final_relu),
        grid=(N // _BLK,),
        in_specs=[
            pl.BlockSpec((_BLK, H), lambda i: (i, 0)),
            pl.BlockSpec((_BLK, H), lambda i: (i, 0)),
            pl.BlockSpec((_BLK, H), lambda i: (i, 0)),
            pl.BlockSpec((H, H), lambda i: (0, 0)),
            pl.BlockSpec((1, H), lambda i: (0, 0)),
            pl.BlockSpec((H, H), lambda i: (0, 0)),
            pl.BlockSpec((1, H), lambda i: (0, 0)),
        ],
        out_specs=pl.BlockSpec((_BLK, H), lambda i: (i, 0)),
        out_shape=jax.ShapeDtypeStruct((N, H), jnp.float32),
    )


_layer2 = _make_layer23(True)
_layer3 = _make_layer23(False)


def _pool_body(h_ref, batch_ref, wf_ref, bf_ref, o_ref):
    gid = lax.broadcasted_iota(jnp.int32, (G, N), 0)
    m = (gid == batch_ref[...]).astype(jnp.float32)
    sums = jnp.dot(m, h_ref[...], preferred_element_type=jnp.float32,
                   precision=_HIGH)
    counts = jnp.sum(m, axis=1, keepdims=True)
    pooled = sums / jnp.maximum(counts, 1.0)
    o_ref[...] = jnp.dot(pooled, wf_ref[...],
                         preferred_element_type=jnp.float32,
                         precision=_HIGH) + bf_ref[...]


_pool = pl.pallas_call(
    _pool_body,
    in_specs=[
        pl.BlockSpec((N, H), lambda: (0, 0)),
        pl.BlockSpec((1, N), lambda: (0, 0)),
        pl.BlockSpec((H, 1), lambda: (0, 0)),
        pl.BlockSpec((1, 1), lambda: (0, 0)),
    ],
    out_specs=pl.BlockSpec((G, 1), lambda: (0, 0)),
    out_shape=jax.ShapeDtypeStruct((G, 1), jnp.float32),
)


def kernel(x, edge_index, batch, W1a, b1a, W1b, b1b, W2a, b2a, W2b, b2b,
           W3a, b3a, W3b, b3b, Wf, bf):
    src_r = edge_index[0].reshape(NUM_CORES, NUM_SUBCORES, CHUNKS, CHUNK)
    dst_r = edge_index[1].reshape(NUM_CORES, NUM_SUBCORES, CHUNKS, CHUNK)
    zeros = jnp.zeros((ROWS_BIG, H), jnp.float32)
    batch2d = batch.reshape(1, N)

    y = _proj(x, W1a)
    p = _agg(y, src_r, dst_r, zeros)
    h1 = _layer1(y, p[0], p[1], b1a.reshape(1, H), W1b, b1b.reshape(1, H))

    p = _agg(h1, src_r, dst_r, zeros)
    h2 = _layer2(h1, p[0], p[1], W2a, b2a.reshape(1, H), W2b, b2b.reshape(1, H))

    p = _agg(h2, src_r, dst_r, zeros)
    h3 = _layer3(h2, p[0], p[1], W3a, b3a.reshape(1, H), W3b, b3b.reshape(1, H))

    return _pool(h3, batch2d, Wf, bf.reshape(1, 1))


# speedup vs baseline: 12.4375x; 1.5240x over previous
import functools

import jax
import jax.numpy as jnp
from jax import lax
from jax.experimental import pallas as pl
from jax.experimental.pallas import tpu as pltpu
from jax.experimental.pallas import tpu_sc as plsc

N, E, F, H, G = 10000, 320000, 128, 64, 128

NUM_CORES = 2
NUM_SUBCORES = 16
NUM_WORKERS = NUM_CORES * NUM_SUBCORES
EDGES_PER_TILE = E // NUM_WORKERS
CHUNK = 80
CHUNKS = EDGES_PER_TILE // CHUNK
NBUF = 5
GROUPS = CHUNKS // NBUF
ROWS_BIG = 632
ROWS_LAST = N - 15 * ROWS_BIG

_HIGH = jax.lax.Precision.HIGHEST


def _agg_body(y_hbm, srcs_hbm, dsts_hbm, zeros_hbm, out_hbm,
              src_v, dst_v, rows_v, acc_sh, gsem, ssem):
    c = lax.axis_index("c")
    s = lax.axis_index("s")
    base = s * ROWS_BIG
    pltpu.sync_copy(srcs_hbm.at[c].at[s], src_v)
    pltpu.sync_copy(dsts_hbm.at[c].at[s], dst_v)

    @pl.when(s < 15)
    def _():
        pltpu.sync_copy(zeros_hbm, acc_sh.at[pl.ds(base, ROWS_BIG)])

    @pl.when(s == 15)
    def _():
        pltpu.sync_copy(zeros_hbm.at[pl.ds(0, ROWS_LAST)],
                        acc_sh.at[pl.ds(15 * ROWS_BIG, ROWS_LAST)])

    plsc.subcore_barrier()

    def step(g, carry):
        gds = []
        for b in range(NBUF):
            j = g * NBUF + b
            gds.append(pltpu.async_copy(y_hbm.at[src_v.at[j]],
                                        rows_v.at[b], gsem.at[b]))
        sds = []
        for b in range(NBUF):
            j = g * NBUF + b
            gds[b].wait()
            sds.append(pltpu.async_copy(rows_v.at[b], acc_sh.at[dst_v.at[j]],
                                        ssem.at[b], add=True))
        for b in range(NBUF):
            sds[b].wait()
        return carry

    lax.fori_loop(0, GROUPS, step, 0)
    plsc.subcore_barrier()

    @pl.when(s < 15)
    def _():
        pltpu.sync_copy(acc_sh.at[pl.ds(base, ROWS_BIG)],
                        out_hbm.at[c].at[pl.ds(base, ROWS_BIG)])

    @pl.when(s == 15)
    def _():
        pltpu.sync_copy(acc_sh.at[pl.ds(15 * ROWS_BIG, ROWS_LAST)],
                        out_hbm.at[c].at[pl.ds(15 * ROWS_BIG, ROWS_LAST)])


_agg = pl.kernel(
    _agg_body,
    out_type=jax.ShapeDtypeStruct((NUM_CORES, N, H), jnp.float32),
    mesh=plsc.VectorSubcoreMesh(core_axis_name="c", subcore_axis_name="s"),
    scratch_types=[
        pltpu.VMEM((CHUNKS, CHUNK), jnp.int32),
        pltpu.VMEM((CHUNKS, CHUNK), jnp.int32),
        pltpu.VMEM((NBUF, CHUNK, H), jnp.float32),
        pltpu.VMEM_SHARED((N, H), jnp.float32),
        pltpu.SemaphoreType.DMA((NBUF,)),
        pltpu.SemaphoreType.DMA((NBUF,)),
    ],
    compiler_params=pltpu.CompilerParams(use_tc_tiling_on_sc=False),
)


_BLK = 2000


def _proj_body(x_ref, w_ref, o_ref):
    o_ref[...] = jnp.dot(x_ref[...], w_ref[...],
                         preferred_element_type=jnp.float32, precision=_HIGH)


_proj = pl.pallas_call(
    _proj_body,
    grid=(N // _BLK,),
    in_specs=[
        pl.BlockSpec((_BLK, F), lambda i: (i, 0)),
        pl.BlockSpec((F, H), lambda i: (0, 0)),
    ],
    out_specs=pl.BlockSpec((_BLK, H), lambda i: (i, 0)),
    out_shape=jax.ShapeDtypeStruct((N, H), jnp.float32),
)


def _layer1_body(y_ref, p0_ref, p1_ref, ba_ref, wb_ref, bb_ref, o_ref):
    t = y_ref[...] + p0_ref[...] + p1_ref[...] + ba_ref[...]
    t = jnp.maximum(t, 0.0)
    t = jnp.dot(t, wb_ref[...], preferred_element_type=jnp.float32,
                precision=_HIGH) + bb_ref[...]
    o_ref[...] = jnp.maximum(t, 0.0)


_layer1 = pl.pallas_call(
    _layer1_body,
    grid=(N // _BLK,),
    in_specs=[
        pl.BlockSpec((_BLK, H), lambda i: (i, 0)),
        pl.BlockSpec((_BLK, H), lambda i: (i, 0)),
        pl.BlockSpec((_BLK, H), lambda i: (i, 0)),
        pl.BlockSpec((1, H), lambda i: (0, 0)),
        pl.BlockSpec((H, H), lambda i: (0, 0)),
        pl.BlockSpec((1, H), lambda i: (0, 0)),
    ],
    out_specs=pl.BlockSpec((_BLK, H), lambda i: (i, 0)),
    out_shape=jax.ShapeDtypeStruct((N, H), jnp.float32),
)


def _layer23_body(h_ref, p0_ref, p1_ref, wa_ref, ba_ref, wb_ref, bb_ref,
                  o_ref, *, final_relu):
    t = h_ref[...] + p0_ref[...] + p1_ref[...]
    t = jnp.dot(t, wa_ref[...], preferred_element_type=jnp.float32,
                precision=_HIGH) + ba_ref[...]
    t = jnp.maximum(t, 0.0)
    t = jnp.dot(t, wb_ref[...], preferred_element_type=jnp.float32,
                precision=_HIGH) + bb_ref[...]
    o_ref[...] = jnp.maximum(t, 0.0) if final_relu else t


def _make_layer23(final_relu):
    return pl.pallas_call(
        functools.partial(_layer23_body, final_relu=final_relu),
        grid=(N // _BLK,),
        in_specs=[
            pl.BlockSpec((_BLK, H), lambda i: (i, 0)),
            pl.BlockSpec((_BLK, H), lambda i: (i, 0)),
            pl.BlockSpec((_BLK, H), lambda i: (i, 0)),
            pl.BlockSpec((H, H), lambda i: (0, 0)),
            pl.BlockSpec((1, H), lambda i: (0, 0)),
            pl.BlockSpec((H, H), lambda i: (0, 0)),
            pl.BlockSpec((1, H), lambda i: (0, 0)),
        ],
        out_specs=pl.BlockSpec((_BLK, H), lambda i: (i, 0)),
        out_shape=jax.ShapeDtypeStruct((N, H), jnp.float32),
    )


_layer2 = _make_layer23(True)
_layer3 = _make_layer23(False)


def _pool_body(h_ref, batch_ref, wf_ref, bf_ref, o_ref):
    gid = lax.broadcasted_iota(jnp.int32, (G, N), 0)
    m = (gid == batch_ref[...]).astype(jnp.float32)
    sums = jnp.dot(m, h_ref[...], preferred_element_type=jnp.float32,
                   precision=_HIGH)
    counts = jnp.sum(m, axis=1, keepdims=True)
    pooled = sums / jnp.maximum(counts, 1.0)
    o_ref[...] = jnp.dot(pooled, wf_ref[...],
                         preferred_element_type=jnp.float32,
                         precision=_HIGH) + bf_ref[...]


_pool = pl.pallas_call(
    _pool_body,
    in_specs=[
        pl.BlockSpec((N, H), lambda: (0, 0)),
        pl.BlockSpec((1, N), lambda: (0, 0)),
        pl.BlockSpec((H, 1), lambda: (0, 0)),
        pl.BlockSpec((1, 1), lambda: (0, 0)),
    ],
    out_specs=pl.BlockSpec((G, 1), lambda: (0, 0)),
    out_shape=jax.ShapeDtypeStruct((G, 1), jnp.float32),
)


def kernel(x, edge_index, batch, W1a, b1a, W1b, b1b, W2a, b2a, W2b, b2b,
           W3a, b3a, W3b, b3b, Wf, bf):
    src_r = edge_index[0].reshape(NUM_CORES, NUM_SUBCORES, CHUNKS, CHUNK)
    dst_r = edge_index[1].reshape(NUM_CORES, NUM_SUBCORES, CHUNKS, CHUNK)
    zeros = jnp.zeros((ROWS_BIG, H), jnp.float32)
    batch2d = batch.reshape(1, N)

    y = _proj(x, W1a)
    p = _agg(y, src_r, dst_r, zeros)
    h1 = _layer1(y, p[0], p[1], b1a.reshape(1, H), W1b, b1b.reshape(1, H))

    p = _agg(h1, src_r, dst_r, zeros)
    h2 = _layer2(h1, p[0], p[1], W2a, b2a.reshape(1, H), W2b, b2b.reshape(1, H))

    p = _agg(h2, src_r, dst_r, zeros)
    h3 = _layer3(h2, p[0], p[1], W3a, b3a.reshape(1, H), W3b, b3b.reshape(1, H))

    return _pool(h3, batch2d, Wf, bf.reshape(1, 1))
